# baseline (speedup 1.0000x reference)
import functools

import jax
import jax.numpy as jnp
from jax import lax
from jax.experimental import pallas as pl
from jax.experimental.pallas import tpu as pltpu
from jax.experimental.pallas import tpu_sc as plsc

VOCAB = 1000000
DIM = 64
ROWS = 16384
COLS = 50
NC, NS = 2, 16
NW = NC * NS
R_W = ROWS // NW
S_BLK = 128
N_S = R_W // S_BLK
N_UNITS = COLS * N_S

_MESH = plsc.VectorSubcoreMesh(
    core_axis_name="c", subcore_axis_name="s", num_cores=NC, num_subcores=NS
)


@functools.partial(
    pl.kernel,
    out_type=jax.ShapeDtypeStruct((COLS, DIM, ROWS), jnp.float32),
    mesh=_MESH,
    scratch_types=[
        pltpu.VMEM((COLS, R_W), jnp.int32),
        pltpu.VMEM((COLS, R_W), jnp.int32),
        pltpu.VMEM((S_BLK, 2 * DIM), jnp.float32),
        pltpu.VMEM((S_BLK, 2 * DIM), jnp.float32),
        pltpu.VMEM((DIM, S_BLK), jnp.float32),
        pltpu.VMEM((DIM, S_BLK), jnp.float32),
        pltpu.SemaphoreType.DMA,
        pltpu.SemaphoreType.DMA,
        pltpu.SemaphoreType.DMA,
        pltpu.SemaphoreType.DMA,
    ],
    compiler_params=pltpu.CompilerParams(needs_layout_passes=False),
)
def _gather_kernel(xt_hbm, w2_hbm, out_hbm, xfull, idx2, gbuf0, gbuf1,
                   obuf0, obuf1, gsem0, gsem1, ssem0, ssem1):
    wid = lax.axis_index("s") * NC + lax.axis_index("c")
    r0 = wid * R_W

    pltpu.sync_copy(xt_hbm.at[:, pl.ds(r0, R_W)], xfull)

    def prep_col(c, _):
        def prep16(t, _):
            v = xfull[c, pl.ds(t * 16, 16)]
            idx2[c, pl.ds(t * 16, 16)] = lax.shift_right_logical(v, 1)
            return 0
        lax.fori_loop(0, R_W // 16, prep16, 0)
        return 0

    lax.fori_loop(0, COLS, prep_col, 0)

    iota = lax.iota(jnp.int32, 16)

    def fire_g(u, gbuf, sem):
        c = u // N_S
        s = lax.rem(u, N_S)
        pltpu.async_copy(
            w2_hbm.at[idx2.at[c, pl.ds(s * S_BLK, S_BLK)]], gbuf, sem
        )

    def wait_g(gbuf, sem):
        pltpu.make_async_copy(w2_hbm.at[pl.ds(0, S_BLK)], gbuf, sem).wait()

    def select(u, gbuf, obuf):
        c = u // N_S
        s = lax.rem(u, N_S)
        for t in range(S_BLK // 16):
            kvec = iota + (16 * t)
            par = lax.bitwise_and(xfull[c, pl.ds(s * S_BLK + 16 * t, 16)], 1)
            colbase = par * DIM

            def drow(d, cols):
                vals = plsc.load_gather(gbuf, [kvec, cols])
                obuf[d, pl.ds(16 * t, 16)] = vals
                return cols + 1

            lax.fori_loop(0, DIM, drow, colbase)

    def fire_s(u, obuf, sem):
        c = u // N_S
        s = lax.rem(u, N_S)
        pltpu.async_copy(
            obuf, out_hbm.at[c, :, pl.ds(r0 + s * S_BLK, S_BLK)], sem
        )

    def wait_s(obuf, sem):
        pltpu.make_async_copy(obuf, out_hbm.at[0, :, pl.ds(0, S_BLK)], sem).wait()

    fire_g(0, gbuf0, gsem0)

    def pair(m, _):
        u = m * 2

        @pl.when(m > 0)
        def _():
            wait_s(obuf1, ssem1)

        fire_g(u + 1, gbuf1, gsem1)
        wait_g(gbuf0, gsem0)

        @pl.when(m > 0)
        def _():
            wait_s(obuf0, ssem0)

        select(u, gbuf0, obuf0)
        fire_s(u, obuf0, ssem0)

        @pl.when(u + 2 < N_UNITS)
        def _():
            fire_g(u + 2, gbuf0, gsem0)

        wait_g(gbuf1, gsem1)
        select(u + 1, gbuf1, obuf1)
        fire_s(u + 1, obuf1, ssem1)
        return 0

    lax.fori_loop(0, N_UNITS // 2, pair, 0)

    wait_s(obuf0, ssem0)
    wait_s(obuf1, ssem1)


def kernel(x, weight):
    w2 = weight.reshape(VOCAB // 2, 2 * DIM)
    out = _gather_kernel(x.T.astype(jnp.int32), w2)
    return out.transpose(2, 0, 1)
